# Initial kernel scaffold; baseline (speedup 1.0000x reference)
import jax
import jax.numpy as jnp
from jax.experimental import pallas as pl


def kernel(obs, emb_table):
    raise NotImplementedError("write your pallas kernel here")


# speedup vs baseline: 3.3697x; 3.3697x over previous
import functools

import jax
import jax.numpy as jnp
from jax import lax
from jax.experimental import pallas as pl
from jax.experimental.pallas import tpu as pltpu
from jax.experimental.pallas import tpu_sc as plsc

_NC = 2
_NS = 16
_NW = _NC * _NS
_C = 128


@functools.lru_cache(maxsize=None)
def _make_gather(B, D, nchunk):
    b_per_w = B // _NW
    mesh = plsc.VectorSubcoreMesh(core_axis_name="c", subcore_axis_name="s")

    @functools.partial(
        pl.kernel,
        mesh=mesh,
        out_type=jax.ShapeDtypeStruct((B, D), jnp.float32),
        scratch_types=[
            pltpu.VMEM((nchunk, _C), jnp.int32),
            pltpu.VMEM((2, _C, D), jnp.float32),
            pltpu.SemaphoreType.DMA,
            pltpu.SemaphoreType.DMA,
        ],
    )
    def k(idx_hbm, table_hbm, out_hbm, idx_v, rows_v, sem0, sem1):
        wid = lax.axis_index("s") * _NC + lax.axis_index("c")
        base = wid * b_per_w
        pltpu.sync_copy(idx_hbm.at[wid], idx_v)
        pltpu.make_async_copy(
            table_hbm.at[idx_v.at[0]], rows_v.at[0], sem0
        ).start()

        def body(i, carry):
            g0 = 2 * i
            g1 = g0 + 1
            pltpu.make_async_copy(
                table_hbm.at[idx_v.at[g1]], rows_v.at[1], sem1
            ).start()
            pltpu.make_async_copy(
                table_hbm.at[idx_v.at[g0]], rows_v.at[0], sem0
            ).wait()
            pltpu.sync_copy(rows_v.at[0], out_hbm.at[pl.ds(base + g0 * _C, _C)])
            g2 = g0 + 2

            @pl.when(g2 < nchunk)
            def _():
                pltpu.make_async_copy(
                    table_hbm.at[idx_v.at[g2]], rows_v.at[0], sem0
                ).start()

            pltpu.make_async_copy(
                table_hbm.at[idx_v.at[g1]], rows_v.at[1], sem1
            ).wait()
            pltpu.sync_copy(rows_v.at[1], out_hbm.at[pl.ds(base + g1 * _C, _C)])
            return carry

        lax.fori_loop(0, nchunk // 2, body, 0)

    return k


def kernel(obs, emb_table):
    batch, fields = obs.shape
    n_cat, d = emb_table.shape
    b_total = batch * fields
    b_per_w = b_total // _NW
    nchunk = b_per_w // _C
    idx = obs.reshape(_NW, nchunk, _C).astype(jnp.int32)
    out = _make_gather(b_total, d, nchunk)(idx, emb_table)
    return out.reshape(batch, fields, d)


# speedup vs baseline: 3.3832x; 1.0040x over previous
import functools

import jax
import jax.numpy as jnp
from jax import lax
from jax.experimental import pallas as pl
from jax.experimental.pallas import tpu as pltpu
from jax.experimental.pallas import tpu_sc as plsc

_NC = 2
_NS = 16
_NW = _NC * _NS
_C = 128


_NBUF = 4
_DEPTH = 3


@functools.lru_cache(maxsize=None)
def _make_gather(B, D, nchunk):
    b_per_w = B // _NW
    mesh = plsc.VectorSubcoreMesh(core_axis_name="c", subcore_axis_name="s")

    @functools.partial(
        pl.kernel,
        mesh=mesh,
        out_type=jax.ShapeDtypeStruct((B, D), jnp.float32),
        scratch_types=[
            pltpu.VMEM((nchunk, _C), jnp.int32),
            pltpu.VMEM((_NBUF, _C, D), jnp.float32),
        ]
        + [pltpu.SemaphoreType.DMA] * (2 * _NBUF),
    )
    def k(idx_hbm, table_hbm, out_hbm, idx_v, rows_v, *sems):
        semg = sems[:_NBUF]
        semw = sems[_NBUF:]
        wid = lax.axis_index("s") * _NC + lax.axis_index("c")
        base = wid * b_per_w
        pltpu.sync_copy(idx_hbm.at[wid], idx_v)

        def gather_start(c, b):
            pltpu.make_async_copy(
                table_hbm.at[idx_v.at[c]], rows_v.at[b], semg[b]
            ).start()

        def gather_wait(c, b):
            pltpu.make_async_copy(
                table_hbm.at[idx_v.at[c]], rows_v.at[b], semg[b]
            ).wait()

        def write_start(c, b):
            pltpu.make_async_copy(
                rows_v.at[b], out_hbm.at[pl.ds(base + c * _C, _C)], semw[b]
            ).start()

        def write_wait(b):
            pltpu.make_async_copy(
                rows_v.at[b], out_hbm.at[pl.ds(base, _C)], semw[b]
            ).wait()

        for c in range(_DEPTH):
            gather_start(c, c)

        def body(i, carry):
            for b in range(_NBUF):
                c = _NBUF * i + b
                gather_wait(c, b)
                write_start(c, b)
                nb = (b + _DEPTH) % _NBUF
                cn = c + _DEPTH

                @pl.when(cn < nchunk)
                def _():
                    @pl.when(cn - _NBUF >= 0)
                    def _():
                        write_wait(nb)

                    gather_start(cn, nb)

            return carry

        lax.fori_loop(0, nchunk // _NBUF, body, 0)
        for b in range(_NBUF):
            write_wait(b)

    return k


def kernel(obs, emb_table):
    batch, fields = obs.shape
    n_cat, d = emb_table.shape
    b_total = batch * fields
    b_per_w = b_total // _NW
    nchunk = b_per_w // _C
    idx = obs.reshape(_NW, nchunk, _C).astype(jnp.int32)
    out = _make_gather(b_total, d, nchunk)(idx, emb_table)
    return out.reshape(batch, fields, d)


# speedup vs baseline: 5.7914x; 1.7118x over previous
import functools

import jax
import jax.numpy as jnp
from jax import lax
from jax.experimental import pallas as pl
from jax.experimental.pallas import tpu as pltpu
from jax.experimental.pallas import tpu_sc as plsc

_NC = 2
_NS = 16
_NW = _NC * _NS
_R = 8
_NBUF = 2


@functools.lru_cache(maxsize=None)
def _make_gather(batch, fields, d):
    rows_w = batch // _NW
    nchunks = rows_w // _R
    mesh = plsc.VectorSubcoreMesh(core_axis_name="c", subcore_axis_name="s")

    @functools.partial(
        pl.kernel,
        mesh=mesh,
        out_type=jax.ShapeDtypeStruct((batch, fields, d), jnp.float32),
        scratch_types=[
            pltpu.VMEM((rows_w, fields), jnp.int32),
            pltpu.VMEM((_NBUF, _R, fields, d), jnp.float32),
        ]
        + [pltpu.SemaphoreType.DMA] * (2 * _NBUF),
    )
    def k(obs_hbm, table_hbm, out_hbm, idx_v, rows_v, *sems):
        semg = sems[:_NBUF]
        semw = sems[_NBUF:]
        wid = lax.axis_index("s") * _NC + lax.axis_index("c")
        row0 = wid * rows_w
        pltpu.sync_copy(obs_hbm.at[pl.ds(row0, rows_w)], idx_v)

        def fire(ch, b):
            def f(r, c):
                pltpu.make_async_copy(
                    table_hbm.at[idx_v.at[ch * _R + r]],
                    rows_v.at[b, r],
                    semg[b],
                ).start()
                return c

            lax.fori_loop(0, _R, f, 0)

        def drain(b):
            def f(r, c):
                pltpu.make_async_copy(
                    table_hbm.at[idx_v.at[0]], rows_v.at[b, r], semg[b]
                ).wait()
                return c

            lax.fori_loop(0, _R, f, 0)

        def write_start(ch, b):
            pltpu.make_async_copy(
                rows_v.at[b], out_hbm.at[pl.ds(row0 + ch * _R, _R)], semw[b]
            ).start()

        def write_wait(b):
            pltpu.make_async_copy(
                rows_v.at[b], out_hbm.at[pl.ds(row0, _R)], semw[b]
            ).wait()

        fire(0, 0)

        def body(i, carry):
            for b in range(_NBUF):
                ch = _NBUF * i + b
                nb = (b + 1) % _NBUF
                cnext = ch + 1

                @pl.when(cnext < nchunks)
                def _():
                    @pl.when(cnext >= _NBUF)
                    def _():
                        write_wait(nb)

                    fire(cnext, nb)

                drain(b)
                write_start(ch, b)
            return carry

        lax.fori_loop(0, nchunks // _NBUF, body, 0)
        for b in range(_NBUF):
            write_wait(b)

    return k


def kernel(obs, emb_table):
    batch, fields = obs.shape
    n_cat, d = emb_table.shape
    return _make_gather(batch, fields, d)(obs.astype(jnp.int32), emb_table)
